# baseline (speedup 1.0000x reference)
import functools

import jax
import jax.numpy as jnp
from jax import lax
from jax.experimental import pallas as pl
from jax.experimental.pallas import tpu as pltpu
from jax.experimental.pallas import tpu_sc as plsc

D = 64
LANES = 128
SGS = 7
IGL = 4


@functools.lru_cache(maxsize=None)
def _make_lookup(N: int, SEQ: int, VP: int):
    info = plsc.get_sparse_core_info()
    NC, NS = info.num_cores, info.num_subcores
    NW = NC * NS
    assert N == NW * IGL * LANES
    n_sub = IGL * SGS * 8
    mesh = plsc.VectorSubcoreMesh(core_axis_name="c", subcore_axis_name="s")

    @functools.partial(
        pl.kernel,
        mesh=mesh,
        out_type=jax.ShapeDtypeStruct((SEQ, D, N), jnp.float32),
        scratch_types=[
            pltpu.VMEM_SHARED((VP, 128), jnp.float32),
            pltpu.VMEM((SGS, 8, IGL * LANES), jnp.int32),
            pltpu.VMEM((LANES, 128), jnp.float32),
            pltpu.VMEM((LANES, 128), jnp.float32),
            pltpu.VMEM((LANES, 128), jnp.float32),
            pltpu.VMEM((LANES, 128), jnp.float32),
            pltpu.VMEM((D, LANES), jnp.float32),
            pltpu.VMEM((D, LANES), jnp.float32),
            pltpu.SemaphoreType.DMA,
            pltpu.SemaphoreType.DMA,
            pltpu.SemaphoreType.DMA,
            pltpu.SemaphoreType.DMA,
            pltpu.SemaphoreType.DMA,
            pltpu.SemaphoreType.DMA,
        ],
        compiler_params=pltpu.CompilerParams(
            use_tc_tiling_on_sc=True, needs_layout_passes=False
        ),
    )
    def k(xp_hbm, tp_hbm, out_hbm, table_sp, idx_v, row0, row1, row2, row3,
          ob0, ob1, g0, g1, g2, g3, o0, o1):
        sid = lax.axis_index("s")
        wid = sid * NC + lax.axis_index("c")
        rows = (row0, row1, row2, row3)
        obufs = (ob0, ob1)
        gsem = (g0, g1, g2, g3)
        osem = (o0, o1)

        @pl.when(sid == 0)
        def _():
            pltpu.sync_copy(tp_hbm, table_sp)

        for sg in range(SGS):
            pltpu.sync_copy(
                xp_hbm.at[pl.ds(sg * 8, 8), pl.ds(wid * (IGL * LANES), IGL * LANES)],
                idx_v.at[sg],
            )
        plsc.subcore_barrier()

        lane = lax.iota(jnp.int32, 16)

        def coords(t):
            igl = t // (SGS * 8)
            sg = (t // 8) % SGS
            sl = t % 8
            s = sg * 8 + sl
            return igl, sg, sl, s

        def valid(t):
            _, _, _, s = coords(t)
            return jnp.logical_and(t < n_sub, s < SEQ)

        def idx_ref(t):
            igl, sg, sl, _ = coords(t)
            return idx_v.at[sg, sl, pl.ds(igl * LANES, LANES)]

        def gather_copy(t, b):
            return pltpu.make_async_copy(
                table_sp.at[idx_ref(t)], rows[b], gsem[b]
            )

        def store_copy(t, b):
            igl, _, _, s = coords(t)
            ig = wid * IGL + igl
            return pltpu.make_async_copy(
                obufs[b], out_hbm.at[s, :, pl.ds(ig * LANES, LANES)], osem[b]
            )

        def transpose2(rb_i, ob_i):
            rb, ob = rows[rb_i], obufs[ob_i]
            cvecs = [c0 * 16 + lane for c0 in range(4)]

            def off_body(off, carry):
                rot = (off + lane) & 15
                for blk in range(8):
                    ilv = rot + blk * 16
                    for c0 in range(4):
                        vals = plsc.load_gather(rb, [ilv, cvecs[c0]])
                        plsc.store_scatter(ob, [cvecs[c0], ilv], vals)
                return carry

            lax.fori_loop(0, 16, off_body, 0)

        for p in range(3):
            @pl.when(valid(p))
            def _():
                gather_copy(p, p).start()

        def body(j, carry):
            for u in range(4):
                t = j * 4 + u
                b = u
                ob = u % 2

                @pl.when(valid(t))
                def _():
                    gather_copy(t, b).wait()

                @pl.when(valid(t + 3))
                def _():
                    gather_copy(t + 3, (u + 3) % 4).start()

                @pl.when(jnp.logical_and(t >= 2, valid(t - 2)))
                def _():
                    store_copy(t - 2, ob).wait()

                @pl.when(valid(t))
                def _():
                    transpose2(b, ob)
                    store_copy(t, ob).start()

            return carry

        lax.fori_loop(0, n_sub // 4, body, 0)

        @pl.when(valid(n_sub - 2))
        def _():
            store_copy(n_sub - 2, 0).wait()

        @pl.when(valid(n_sub - 1))
        def _():
            store_copy(n_sub - 1, 1).wait()

    return k


def kernel(x, table):
    n, seq = x.shape
    xp = jnp.pad(x.T.astype(jnp.int32), ((0, SGS * 8 - seq), (0, 0)))
    tp = jnp.pad(table.astype(jnp.float32),
                 ((0, 7), (0, 128 - table.shape[1])))
    out = _make_lookup(n, seq, tp.shape[0])(xp, tp)
    return jnp.transpose(out, (2, 0, 1))


# speedup vs baseline: 2.3011x; 2.3011x over previous
import functools

import jax
import jax.numpy as jnp
from jax import lax
from jax.experimental import pallas as pl
from jax.experimental.pallas import tpu as pltpu
from jax.experimental.pallas import tpu_sc as plsc

D = 64
LANES = 128
SGS = 7
IGL = 4


@functools.lru_cache(maxsize=None)
def _make_lookup(N: int, SEQ: int, VP: int):
    info = plsc.get_sparse_core_info()
    NC, NS = info.num_cores, info.num_subcores
    NW = NC * NS
    assert N == NW * IGL * LANES
    n_sub = IGL * SGS * 8
    mesh = plsc.VectorSubcoreMesh(core_axis_name="c", subcore_axis_name="s")

    @functools.partial(
        pl.kernel,
        mesh=mesh,
        out_type=jax.ShapeDtypeStruct((SEQ, D, N), jnp.float32),
        scratch_types=[
            pltpu.VMEM_SHARED((VP, 128), jnp.float32),
            pltpu.VMEM((SGS, 8, IGL * LANES), jnp.int32),
            pltpu.VMEM((LANES, 128), jnp.float32),
            pltpu.VMEM((LANES, 128), jnp.float32),
            pltpu.VMEM((LANES, 128), jnp.float32),
            pltpu.VMEM((LANES, 128), jnp.float32),
            pltpu.VMEM((D, LANES), jnp.float32),
            pltpu.VMEM((D, LANES), jnp.float32),
            pltpu.SemaphoreType.DMA,
            pltpu.SemaphoreType.DMA,
            pltpu.SemaphoreType.DMA,
            pltpu.SemaphoreType.DMA,
            pltpu.SemaphoreType.DMA,
            pltpu.SemaphoreType.DMA,
        ],
        compiler_params=pltpu.CompilerParams(
            use_tc_tiling_on_sc=True, needs_layout_passes=False
        ),
    )
    def k(xp_hbm, tp_hbm, out_hbm, table_sp, idx_v, row0, row1, row2, row3,
          ob0, ob1, g0, g1, g2, g3, o0, o1):
        sid = lax.axis_index("s")
        wid = sid * NC + lax.axis_index("c")
        rows = (row0, row1, row2, row3)
        obufs = (ob0, ob1)
        gsem = (g0, g1, g2, g3)
        osem = (o0, o1)

        @pl.when(sid == 0)
        def _():
            pltpu.sync_copy(tp_hbm, table_sp)

        for sg in range(SGS):
            pltpu.sync_copy(
                xp_hbm.at[pl.ds(sg * 8, 8), pl.ds(wid * (IGL * LANES), IGL * LANES)],
                idx_v.at[sg],
            )
        plsc.subcore_barrier()

        lane = lax.iota(jnp.int32, 16)

        def coords(t):
            igl = t // (SGS * 8)
            sg = (t // 8) % SGS
            sl = t % 8
            s = sg * 8 + sl
            return igl, sg, sl, s

        def valid(t):
            _, _, _, s = coords(t)
            return jnp.logical_and(t < n_sub, s < SEQ)

        def idx_ref(t):
            igl, sg, sl, _ = coords(t)
            return idx_v.at[sg, sl, pl.ds(igl * LANES, LANES)]

        def gather_copy(t, b):
            return pltpu.make_async_copy(
                table_sp.at[idx_ref(t)], rows[b], gsem[b]
            )

        def store_copy(t, b):
            igl, _, _, s = coords(t)
            ig = wid * IGL + igl
            return pltpu.make_async_copy(
                obufs[b], out_hbm.at[s, :, pl.ds(ig * LANES, LANES)], osem[b]
            )

        def transpose2(rb_i, ob_i):
            rb, ob = rows[rb_i], obufs[ob_i]
            cvecs = [c0 * 16 + lane for c0 in range(4)]

            @plsc.parallel_loop(0, 128, unroll=4)
            def off_body(i):
                off = i & 15
                blk = i >> 4
                ilv = ((off + lane) & 15) + blk * 16
                for c0 in range(4):
                    vals = plsc.load_gather(rb, [ilv, cvecs[c0]])
                    plsc.store_scatter(ob, [cvecs[c0], ilv], vals)

        for p in range(3):
            @pl.when(valid(p))
            def _():
                gather_copy(p, p).start()

        def body(j, carry):
            for u in range(4):
                t = j * 4 + u
                b = u
                ob = u % 2

                @pl.when(valid(t))
                def _():
                    gather_copy(t, b).wait()

                @pl.when(valid(t + 3))
                def _():
                    gather_copy(t + 3, (u + 3) % 4).start()

                @pl.when(jnp.logical_and(t >= 2, valid(t - 2)))
                def _():
                    store_copy(t - 2, ob).wait()

                @pl.when(valid(t))
                def _():
                    transpose2(b, ob)
                    store_copy(t, ob).start()

            return carry

        lax.fori_loop(0, n_sub // 4, body, 0)

        @pl.when(valid(n_sub - 2))
        def _():
            store_copy(n_sub - 2, 0).wait()

        @pl.when(valid(n_sub - 1))
        def _():
            store_copy(n_sub - 1, 1).wait()

    return k


def kernel(x, table):
    n, seq = x.shape
    xp = jnp.pad(x.T.astype(jnp.int32), ((0, SGS * 8 - seq), (0, 0)))
    tp = jnp.pad(table.astype(jnp.float32),
                 ((0, 7), (0, 128 - table.shape[1])))
    out = _make_lookup(n, seq, tp.shape[0])(xp, tp)
    return jnp.transpose(out, (2, 0, 1))
